# baseline (speedup 1.0000x reference)
import functools

import jax
import jax.numpy as jnp
from jax import lax
from jax.experimental import pallas as pl
from jax.experimental.pallas import tpu as pltpu
from jax.experimental.pallas import tpu_sc as plsc

B, S, H = 4096, 200, 64
NC, NS = 2, 16
NW = NC * NS
BPW = B // NW
JB = BPW // 16

_mesh = plsc.VectorSubcoreMesh(core_axis_name="c", subcore_axis_name="s")


@functools.partial(
    pl.kernel,
    out_type=jax.ShapeDtypeStruct((B, H), jnp.float32),
    mesh=_mesh,
    scratch_types=[
        pltpu.VMEM((S, BPW), jnp.int32),
        pltpu.VMEM((BPW, H), jnp.float32),
        pltpu.VMEM((1, H), jnp.float32),
        pltpu.VMEM((BPW,), jnp.float32),
        pltpu.SemaphoreType.DMA,
    ],
    compiler_params=pltpu.CompilerParams(use_tc_tiling_on_sc=False),
)
def _embed_sum(seqs_t_hbm, weight_hbm, out_hbm, idx_v, acc_v, w0_v, cnt_v,
               sem):
    wid = lax.axis_index("s") * NC + lax.axis_index("c")
    base = wid * BPW
    pltpu.sync_copy(seqs_t_hbm.at[:, pl.ds(base, BPW)], idx_v)
    pltpu.sync_copy(weight_hbm.at[pl.ds(0, 1)], w0_v)

    zero = jnp.zeros((16,), jnp.float32)

    def zrow(b, carry):
        for k in range(H // 16):
            acc_v[b, pl.ds(k * 16, 16)] = zero
        return carry

    lax.fori_loop(0, BPW, zrow, 0)

    def fire(t, carry):
        pltpu.async_copy(weight_hbm.at[idx_v.at[t]], acc_v, sem, add=True)
        return carry

    lax.fori_loop(0, S, fire, 0)

    one = jnp.ones((16,), jnp.float32)

    def count(t, cnts):
        return tuple(
            cnts[j] + jnp.where(idx_v[t, pl.ds(j * 16, 16)] == 0, one, zero)
            for j in range(JB))

    cnts = lax.fori_loop(0, S, count, (zero,) * JB)

    def drain(t, carry):
        pltpu.make_async_copy(weight_hbm.at[idx_v.at[t]], acc_v, sem).wait()
        return carry

    lax.fori_loop(0, S, drain, 0)

    w0 = [w0_v[0, pl.ds(k * 16, 16)] for k in range(H // 16)]
    for j in range(JB):
        for i in range(16):
            b = j * 16 + i
            n0 = cnts[j][i]
            for k in range(H // 16):
                sl = pl.ds(k * 16, 16)
                acc_v[b, sl] = acc_v[b, sl] - n0 * w0[k]
    pltpu.sync_copy(acc_v, out_hbm.at[pl.ds(base, BPW)])


def _transpose_body(x_ref, o_ref):
    o_ref[...] = x_ref[...].T


_transpose = pl.pallas_call(
    _transpose_body,
    out_shape=jax.ShapeDtypeStruct((S, B), jnp.int32),
)


def kernel(seqs, weight):
    return _embed_sum(_transpose(seqs), weight)


# speedup vs baseline: 1.6134x; 1.6134x over previous
import functools

import jax
import jax.numpy as jnp
from jax import lax
from jax.experimental import pallas as pl
from jax.experimental.pallas import tpu as pltpu
from jax.experimental.pallas import tpu_sc as plsc

B, S, H = 4096, 200, 64
NC, NS = 2, 16
NW = NC * NS
BPW = B // NW
JB = BPW // 16

_mesh = plsc.VectorSubcoreMesh(core_axis_name="c", subcore_axis_name="s")


@functools.partial(
    pl.kernel,
    out_type=jax.ShapeDtypeStruct((B, H), jnp.float32),
    mesh=_mesh,
    scratch_types=[
        pltpu.VMEM((S, BPW), jnp.int32),
        pltpu.VMEM((BPW, H), jnp.float32),
        pltpu.VMEM((1, H), jnp.float32),
        pltpu.VMEM((BPW,), jnp.float32),
        pltpu.SemaphoreType.DMA,
    ],
    compiler_params=pltpu.CompilerParams(use_tc_tiling_on_sc=False),
)
def _embed_sum(seqs_t_hbm, weight_hbm, out_hbm, idx_v, acc_v, w0_v, cnt_v,
               sem):
    wid = lax.axis_index("s") * NC + lax.axis_index("c")
    base = wid * BPW
    pltpu.sync_copy(seqs_t_hbm.at[:, pl.ds(base, BPW)], idx_v)
    pltpu.sync_copy(weight_hbm.at[pl.ds(0, 1)], w0_v)

    zero = jnp.zeros((16,), jnp.float32)

    def zrow(b, carry):
        for k in range(H // 16):
            acc_v[b, pl.ds(k * 16, 16)] = zero
        return carry

    lax.fori_loop(0, BPW, zrow, 0)

    def fire(t, carry):
        pltpu.async_copy(weight_hbm.at[idx_v.at[t]], acc_v, sem, add=True)
        return carry

    lax.fori_loop(0, S, fire, 0)

    one = jnp.ones((16,), jnp.float32)

    def count(t, cnts):
        return tuple(
            cnts[j] + jnp.where(idx_v[t, pl.ds(j * 16, 16)] == 0, one, zero)
            for j in range(JB))

    cnts = lax.fori_loop(0, S, count, (zero,) * JB)

    def drain(t, carry):
        pltpu.make_async_copy(weight_hbm.at[idx_v.at[t]], acc_v, sem).wait()
        return carry

    lax.fori_loop(0, S, drain, 0)

    w0 = [w0_v[0, pl.ds(k * 16, 16)] for k in range(H // 16)]
    for j in range(JB):
        for i in range(16):
            b = j * 16 + i
            n0 = cnts[j][i]
            for k in range(H // 16):
                sl = pl.ds(k * 16, 16)
                acc_v[b, sl] = acc_v[b, sl] - n0 * w0[k]
    pltpu.sync_copy(acc_v, out_hbm.at[pl.ds(base, BPW)])


VOCAB = 1000000
VCHUNK = 4096
VGRID = -(-VOCAB // VCHUNK)
VTAIL = VOCAB - (VGRID - 1) * VCHUNK
PR = (VGRID - 1) * (VCHUNK // 2) + VTAIL
V2 = 2 * PR


def _transpose_body(x_ref, o_ref):
    v = x_ref[...]
    r = v & (VCHUNK - 1)
    pi = (v - r) + 2 * (r & (VCHUNK // 2 - 1)) + (r >> 11)
    o_ref[...] = pi.T


_transpose = pl.pallas_call(
    _transpose_body,
    out_shape=jax.ShapeDtypeStruct((S, B), jnp.int32),
)


def _relayout_body(x_ref, o_ref):
    y = x_ref[...].T
    o_ref[...] = jnp.concatenate(
        [y[0:VCHUNK // 2, :], y[VCHUNK // 2:VCHUNK, :]], axis=1)


_relayout = pl.pallas_call(
    _relayout_body,
    grid=(VGRID,),
    in_specs=[pl.BlockSpec((H, VCHUNK), lambda i: (0, i))],
    out_specs=pl.BlockSpec((VCHUNK // 2, 2 * H), lambda i: (i, 0)),
    out_shape=jax.ShapeDtypeStruct((PR, 2 * H), jnp.float32),
)


def kernel(seqs, weight):
    w_rm = _relayout(weight.T).reshape(V2, H)
    return _embed_sum(_transpose(seqs), w_rm)


# speedup vs baseline: 1.9155x; 1.1873x over previous
import functools

import jax
import jax.numpy as jnp
from jax import lax
from jax.experimental import pallas as pl
from jax.experimental.pallas import tpu as pltpu
from jax.experimental.pallas import tpu_sc as plsc

B, S, H = 4096, 200, 64
NC, NS = 2, 16
NW = NC * NS
BPW = B // NW
JB = BPW // 16

_mesh = plsc.VectorSubcoreMesh(core_axis_name="c", subcore_axis_name="s")


@functools.partial(
    pl.kernel,
    out_type=jax.ShapeDtypeStruct((B, H), jnp.float32),
    mesh=_mesh,
    scratch_types=[
        pltpu.VMEM((S, BPW), jnp.int32),
        pltpu.VMEM((BPW, H), jnp.float32),
        pltpu.VMEM((1, H), jnp.float32),
        pltpu.VMEM((BPW,), jnp.float32),
        pltpu.SemaphoreType.DMA,
    ],
    compiler_params=pltpu.CompilerParams(use_tc_tiling_on_sc=False),
)
def _embed_sum(seqs_t_hbm, weight_hbm, out_hbm, idx_v, acc_v, w0_v, cnt_v,
               sem):
    wid = lax.axis_index("s") * NC + lax.axis_index("c")
    base = wid * BPW
    pltpu.sync_copy(seqs_t_hbm.at[:, pl.ds(base, BPW)], idx_v)
    pltpu.sync_copy(weight_hbm.at[pl.ds(0, 1)], w0_v)

    zero = jnp.zeros((16,), jnp.float32)

    def zrow(b, carry):
        for k in range(H // 16):
            acc_v[b, pl.ds(k * 16, 16)] = zero
        return carry

    lax.fori_loop(0, BPW, zrow, 0)

    def fire(t, carry):
        pltpu.async_copy(weight_hbm.at[idx_v.at[t]], acc_v, sem, add=True)
        return carry

    lax.fori_loop(0, S, fire, 0)

    one = jnp.ones((16,), jnp.float32)

    def count(t, cnts):
        return tuple(
            cnts[j] + jnp.where(idx_v[t, pl.ds(j * 16, 16)] == 0, one, zero)
            for j in range(JB))

    cnts = lax.fori_loop(0, S, count, (zero,) * JB)

    def drain(t, carry):
        pltpu.make_async_copy(weight_hbm.at[idx_v.at[t]], acc_v, sem).wait()
        return carry

    lax.fori_loop(0, S, drain, 0)

    w0 = [w0_v[0, pl.ds(k * 16, 16)] for k in range(H // 16)]
    for j in range(JB):
        for i in range(16):
            b = j * 16 + i
            n0 = cnts[j][i]
            for k in range(H // 16):
                sl = pl.ds(k * 16, 16)
                acc_v[b, sl] = acc_v[b, sl] - n0 * w0[k]
    pltpu.sync_copy(acc_v, out_hbm.at[pl.ds(base, BPW)])


VOCAB = 1000000
VCHUNK = 8192
VGRID = -(-VOCAB // VCHUNK)
VTAIL = VOCAB - (VGRID - 1) * VCHUNK
PR = (VGRID - 1) * (VCHUNK // 2) + VTAIL
V2 = 2 * PR


def _transpose_body(x_ref, o_ref):
    v = x_ref[...]
    r = v & (VCHUNK - 1)
    pi = (v - r) + 2 * (r & (VCHUNK // 2 - 1)) + (r >> 12)
    o_ref[...] = pi.T


_transpose = pl.pallas_call(
    _transpose_body,
    out_shape=jax.ShapeDtypeStruct((S, B), jnp.int32),
)


def _relayout_body(x_ref, o_ref):
    y = x_ref[...].T
    o_ref[...] = jnp.concatenate(
        [y[0:VCHUNK // 2, :], y[VCHUNK // 2:VCHUNK, :]], axis=1)


_relayout = pl.pallas_call(
    _relayout_body,
    grid=(VGRID,),
    in_specs=[pl.BlockSpec((H, VCHUNK), lambda i: (0, i))],
    out_specs=pl.BlockSpec((VCHUNK // 2, 2 * H), lambda i: (i, 0)),
    out_shape=jax.ShapeDtypeStruct((PR, 2 * H), jnp.float32),
)


def kernel(seqs, weight):
    w_rm = _relayout(weight.T).reshape(V2, H)
    return _embed_sum(_transpose(seqs), w_rm)


# speedup vs baseline: 2.1033x; 1.0980x over previous
import functools

import jax
import jax.numpy as jnp
from jax import lax
from jax.experimental import pallas as pl
from jax.experimental.pallas import tpu as pltpu
from jax.experimental.pallas import tpu_sc as plsc

B, S, H = 4096, 200, 64
NC, NS = 2, 16
NW = NC * NS
BPW = B // NW
JB = BPW // 16

_mesh = plsc.VectorSubcoreMesh(core_axis_name="c", subcore_axis_name="s")


@functools.partial(
    pl.kernel,
    out_type=jax.ShapeDtypeStruct((B, H), jnp.float32),
    mesh=_mesh,
    scratch_types=[
        pltpu.VMEM((S, BPW), jnp.int32),
        pltpu.VMEM((BPW, H), jnp.float32),
        pltpu.VMEM((1, H), jnp.float32),
        pltpu.VMEM((BPW,), jnp.float32),
        pltpu.SemaphoreType.DMA,
    ],
    compiler_params=pltpu.CompilerParams(use_tc_tiling_on_sc=False),
)
def _embed_sum(seqs_t_hbm, weight_hbm, out_hbm, idx_v, acc_v, w0_v, cnt_v,
               sem):
    wid = lax.axis_index("s") * NC + lax.axis_index("c")
    base = wid * BPW
    pltpu.sync_copy(seqs_t_hbm.at[:, pl.ds(base, BPW)], idx_v)
    pltpu.sync_copy(weight_hbm.at[pl.ds(0, 1)], w0_v)

    zero = jnp.zeros((16,), jnp.float32)

    def zrow(b, carry):
        for k in range(H // 16):
            acc_v[b, pl.ds(k * 16, 16)] = zero
        return carry

    lax.fori_loop(0, BPW, zrow, 0)

    def fire(t, carry):
        pltpu.async_copy(weight_hbm.at[idx_v.at[t]], acc_v, sem, add=True)
        return carry

    lax.fori_loop(0, S, fire, 0)

    one = jnp.ones((16,), jnp.float32)

    def count(t, cnts):
        return tuple(
            cnts[j] + jnp.where(idx_v[t, pl.ds(j * 16, 16)] == 0, one, zero)
            for j in range(JB))

    cnts = lax.fori_loop(0, S, count, (zero,) * JB)

    def drain(t, carry):
        pltpu.make_async_copy(weight_hbm.at[idx_v.at[t]], acc_v, sem).wait()
        return carry

    lax.fori_loop(0, S, drain, 0)

    w0 = [w0_v[0, pl.ds(k * 16, 16)] for k in range(H // 16)]
    for j in range(JB):
        for i in range(16):
            b = j * 16 + i
            n0 = cnts[j][i]
            for k in range(H // 16):
                sl = pl.ds(k * 16, 16)
                acc_v[b, sl] = acc_v[b, sl] - n0 * w0[k]
    pltpu.sync_copy(acc_v, out_hbm.at[pl.ds(base, BPW)])


VOCAB = 1000000
VCHUNK = 16384
VGRID = -(-VOCAB // VCHUNK)
VTAIL = VOCAB - (VGRID - 1) * VCHUNK
PR = (VGRID - 1) * (VCHUNK // 2) + VTAIL
V2 = 2 * PR


def _transpose_body(x_ref, o_ref):
    v = x_ref[...]
    r = v & (VCHUNK - 1)
    pi = (v - r) + 2 * (r & (VCHUNK // 2 - 1)) + (r >> 13)
    o_ref[...] = pi.T


_transpose = pl.pallas_call(
    _transpose_body,
    out_shape=jax.ShapeDtypeStruct((S, B), jnp.int32),
)


def _relayout_body(x_ref, o_ref):
    y = x_ref[...].T
    o_ref[...] = jnp.concatenate(
        [y[0:VCHUNK // 2, :], y[VCHUNK // 2:VCHUNK, :]], axis=1)


_relayout = pl.pallas_call(
    _relayout_body,
    grid=(VGRID,),
    in_specs=[pl.BlockSpec((H, VCHUNK), lambda i: (0, i))],
    out_specs=pl.BlockSpec((VCHUNK // 2, 2 * H), lambda i: (i, 0)),
    out_shape=jax.ShapeDtypeStruct((PR, 2 * H), jnp.float32),
    compiler_params=pltpu.CompilerParams(vmem_limit_bytes=100 * 1024 * 1024),
)


def kernel(seqs, weight):
    w_rm = _relayout(weight.T).reshape(V2, H)
    return _embed_sum(_transpose(seqs), w_rm)


# speedup vs baseline: 2.1965x; 1.0443x over previous
import functools

import jax
import jax.numpy as jnp
from jax import lax
from jax.experimental import pallas as pl
from jax.experimental.pallas import tpu as pltpu
from jax.experimental.pallas import tpu_sc as plsc

B, S, H = 4096, 200, 64
NC, NS = 2, 16
NW = NC * NS
BPW = B // NW
JB = BPW // 16

_mesh = plsc.VectorSubcoreMesh(core_axis_name="c", subcore_axis_name="s")


@functools.partial(
    pl.kernel,
    out_type=jax.ShapeDtypeStruct((B, H), jnp.float32),
    mesh=_mesh,
    scratch_types=[
        pltpu.VMEM((S, BPW), jnp.int32),
        pltpu.VMEM((BPW, H), jnp.float32),
        pltpu.VMEM((1, H), jnp.float32),
        pltpu.VMEM((BPW,), jnp.float32),
        pltpu.SemaphoreType.DMA,
    ],
    compiler_params=pltpu.CompilerParams(use_tc_tiling_on_sc=False),
)
def _embed_sum(seqs_t_hbm, weight_hbm, out_hbm, idx_v, acc_v, w0_v, cnt_v,
               sem):
    wid = lax.axis_index("s") * NC + lax.axis_index("c")
    base = wid * BPW
    pltpu.sync_copy(seqs_t_hbm.at[:, pl.ds(base, BPW)], idx_v)
    pltpu.sync_copy(weight_hbm.at[pl.ds(0, 1)], w0_v)

    zero = jnp.zeros((16,), jnp.float32)

    def zrow(b, carry):
        for k in range(H // 16):
            acc_v[b, pl.ds(k * 16, 16)] = zero
        return carry

    lax.fori_loop(0, BPW, zrow, 0)

    def fire(t, carry):
        pltpu.async_copy(weight_hbm.at[idx_v.at[t]], acc_v, sem, add=True)
        return carry

    lax.fori_loop(0, S, fire, 0)

    one = jnp.ones((16,), jnp.float32)

    def count(t, cnts):
        return tuple(
            cnts[j] + jnp.where(idx_v[t, pl.ds(j * 16, 16)] == 0, one, zero)
            for j in range(JB))

    cnts = lax.fori_loop(0, S, count, (zero,) * JB)

    def drain(t, carry):
        pltpu.make_async_copy(weight_hbm.at[idx_v.at[t]], acc_v, sem).wait()
        return carry

    lax.fori_loop(0, S, drain, 0)

    w0 = [w0_v[0, pl.ds(k * 16, 16)] for k in range(H // 16)]
    for j in range(JB):
        for i in range(16):
            b = j * 16 + i
            n0 = cnts[j][i]
            for k in range(H // 16):
                sl = pl.ds(k * 16, 16)
                acc_v[b, sl] = acc_v[b, sl] - n0 * w0[k]
    pltpu.sync_copy(acc_v, out_hbm.at[pl.ds(base, BPW)])


VOCAB = 1000000
VCHUNK = 32768
VGRID = -(-VOCAB // VCHUNK)
VTAIL = VOCAB - (VGRID - 1) * VCHUNK
PR = (VGRID - 1) * (VCHUNK // 2) + min(VTAIL, VCHUNK // 2)
V2 = 2 * PR


def _transpose_body(x_ref, o_ref):
    v = x_ref[...]
    r = v & (VCHUNK - 1)
    pi = (v - r) + 2 * (r & (VCHUNK // 2 - 1)) + (r >> 14)
    o_ref[...] = pi.T


_transpose = pl.pallas_call(
    _transpose_body,
    out_shape=jax.ShapeDtypeStruct((S, B), jnp.int32),
)


def _relayout_body(x_ref, o_ref):
    y = x_ref[...].T
    o_ref[...] = jnp.concatenate(
        [y[0:VCHUNK // 2, :], y[VCHUNK // 2:VCHUNK, :]], axis=1)


_relayout = pl.pallas_call(
    _relayout_body,
    grid=(VGRID,),
    in_specs=[pl.BlockSpec((H, VCHUNK), lambda i: (0, i))],
    out_specs=pl.BlockSpec((VCHUNK // 2, 2 * H), lambda i: (i, 0)),
    out_shape=jax.ShapeDtypeStruct((PR, 2 * H), jnp.float32),
    compiler_params=pltpu.CompilerParams(vmem_limit_bytes=100 * 1024 * 1024),
)


def kernel(seqs, weight):
    w_rm = _relayout(weight.T).reshape(V2, H)
    return _embed_sum(_transpose(seqs), w_rm)


# speedup vs baseline: 2.1982x; 1.0008x over previous
import functools

import jax
import jax.numpy as jnp
from jax import lax
from jax.experimental import pallas as pl
from jax.experimental.pallas import tpu as pltpu
from jax.experimental.pallas import tpu_sc as plsc

B, S, H = 4096, 200, 64
NC, NS = 2, 16
NW = NC * NS
BPW = B // NW
JB = BPW // 16

_mesh = plsc.VectorSubcoreMesh(core_axis_name="c", subcore_axis_name="s")


@functools.partial(
    pl.kernel,
    out_type=jax.ShapeDtypeStruct((B, H), jnp.float32),
    mesh=_mesh,
    scratch_types=[
        pltpu.VMEM((S, BPW), jnp.int32),
        pltpu.VMEM((BPW, H), jnp.float32),
        pltpu.VMEM((1, H), jnp.float32),
        pltpu.VMEM((BPW,), jnp.float32),
        pltpu.SemaphoreType.DMA,
    ],
    compiler_params=pltpu.CompilerParams(use_tc_tiling_on_sc=False),
)
def _embed_sum(seqs_t_hbm, weight_hbm, out_hbm, idx_v, acc_v, w0_v, cnt_v,
               sem):
    wid = lax.axis_index("s") * NC + lax.axis_index("c")
    base = wid * BPW
    pltpu.sync_copy(seqs_t_hbm.at[:, pl.ds(base, BPW)], idx_v)
    pltpu.sync_copy(weight_hbm.at[pl.ds(0, 1)], w0_v)

    zero = jnp.zeros((16,), jnp.float32)

    def zrow(b, carry):
        for k in range(H // 16):
            acc_v[b, pl.ds(k * 16, 16)] = zero
        return carry

    lax.fori_loop(0, BPW, zrow, 0)

    def fire(t, carry):
        pltpu.async_copy(weight_hbm.at[idx_v.at[t]], acc_v, sem, add=True)
        return carry

    lax.fori_loop(0, S, fire, 0)

    one = jnp.ones((16,), jnp.float32)

    def count(t, cnts):
        return tuple(
            cnts[j] + jnp.where(idx_v[t, pl.ds(j * 16, 16)] == 0, one, zero)
            for j in range(JB))

    cnts = lax.fori_loop(0, S, count, (zero,) * JB)

    def drain(t, carry):
        pltpu.make_async_copy(weight_hbm.at[idx_v.at[t]], acc_v, sem).wait()
        return carry

    lax.fori_loop(0, S, drain, 0)

    w0 = [w0_v[0, pl.ds(k * 16, 16)] for k in range(H // 16)]
    for j in range(JB):
        for i in range(16):
            b = j * 16 + i
            n0 = cnts[j][i]
            for k in range(H // 16):
                sl = pl.ds(k * 16, 16)
                acc_v[b, sl] = acc_v[b, sl] - n0 * w0[k]
    pltpu.sync_copy(acc_v, out_hbm.at[pl.ds(base, BPW)])


VOCAB = 1000000
VCHUNK = 32768
VGRID = -(-VOCAB // VCHUNK)
VTAIL = VOCAB - (VGRID - 1) * VCHUNK
PR = (VGRID - 1) * (VCHUNK // 2) + min(VTAIL, VCHUNK // 2)
V2 = 2 * PR
_HSHIFT = (VCHUNK // 2).bit_length() - 1


def _transpose_body(x_ref, o_ref):
    v = x_ref[...]
    r = v & (VCHUNK - 1)
    pi = (v - r) + 2 * (r & (VCHUNK // 2 - 1)) + (r >> _HSHIFT)
    o_ref[...] = pi.T


_transpose = pl.pallas_call(
    _transpose_body,
    out_shape=jax.ShapeDtypeStruct((S, B), jnp.int32),
)


def _relayout_body(x_ref, o_ref):
    y = x_ref[...].T
    o_ref[...] = jnp.concatenate(
        [y[0:VCHUNK // 2, :], y[VCHUNK // 2:VCHUNK, :]], axis=1)


_relayout = pl.pallas_call(
    _relayout_body,
    grid=(VGRID,),
    in_specs=[pl.BlockSpec((H, VCHUNK), lambda i: (0, i))],
    out_specs=pl.BlockSpec((VCHUNK // 2, 2 * H), lambda i: (i, 0)),
    out_shape=jax.ShapeDtypeStruct((PR, 2 * H), jnp.float32),
    compiler_params=pltpu.CompilerParams(vmem_limit_bytes=100 * 1024 * 1024),
)


def kernel(seqs, weight):
    w_rm = _relayout(weight.T).reshape(V2, H)
    return _embed_sum(_transpose(seqs), w_rm)
